# Initial kernel scaffold; baseline (speedup 1.0000x reference)
import jax
import jax.numpy as jnp
from jax.experimental import pallas as pl


def kernel(x, table, W, b):
    raise NotImplementedError("write your pallas kernel here")


# speedup vs baseline: 23.3656x; 23.3656x over previous
import functools

import jax
import jax.numpy as jnp
from jax import lax
from jax.experimental import pallas as pl
from jax.experimental.pallas import tpu as pltpu
from jax.experimental.pallas import tpu_sc as plsc

EMB = 1_000_000
H1 = 16
BATCH = 16384
SEQ = 50
TOKENS = BATCH * SEQ

NC = 2
NS = 16
NW = NC * NS

TOK_PER_W = TOKENS // NW
CHUNK = 128
NCHUNK = TOK_PER_W // CHUNK


def _gather_body(idx_hbm, table_hbm, out_hbm, idx_v, rows_v, sem):
    wid = lax.axis_index("c") * NS + lax.axis_index("s")
    pltpu.sync_copy(idx_hbm.at[pl.ds(wid * NCHUNK, NCHUNK)], idx_v)

    def step(j, carry):
        buf = jax.lax.rem(j, 2)
        pltpu.async_copy(table_hbm.at[idx_v.at[j]], rows_v.at[buf], sem).wait()
        tok = wid * TOK_PER_W + j * CHUNK
        pltpu.sync_copy(rows_v.at[buf], out_hbm.at[pl.ds(tok, CHUNK)])
        return carry

    lax.fori_loop(0, NCHUNK, step, 0)


_gather = functools.partial(
    pl.kernel,
    out_type=jax.ShapeDtypeStruct((TOKENS, H1), jnp.float32),
    scratch_types=[
        pltpu.VMEM((NCHUNK, CHUNK), jnp.int32),
        pltpu.VMEM((2, CHUNK, H1), jnp.float32),
        pltpu.SemaphoreType.DMA,
    ],
    mesh=plsc.VectorSubcoreMesh(core_axis_name="c", subcore_axis_name="s"),
    compiler_params=pltpu.CompilerParams(use_tc_tiling_on_sc=False),
)(_gather_body)


def _dense_body(e_ref, w_ref, b_ref, o_ref):
    logits = jnp.dot(e_ref[...], w_ref[...],
                     preferred_element_type=jnp.float32) + b_ref[...]
    m = jnp.max(logits, axis=-1, keepdims=True)
    s = jnp.sum(jnp.exp(logits - m), axis=-1, keepdims=True)
    o_ref[...] = logits - m - jnp.log(s)


def kernel(x, table, W, b):
    idx = x.reshape(NW * NCHUNK, CHUNK)
    embeds = _gather(idx, table)
    e2 = embeds.reshape(BATCH, SEQ * H1)

    bm = 2048
    out = pl.pallas_call(
        _dense_body,
        grid=(BATCH // bm,),
        in_specs=[
            pl.BlockSpec((bm, SEQ * H1), lambda i: (i, 0)),
            pl.BlockSpec((SEQ * H1, 3), lambda i: (0, 0)),
            pl.BlockSpec((1, 3), lambda i: (0, 0)),
        ],
        out_specs=pl.BlockSpec((bm, 3), lambda i: (i, 0)),
        out_shape=jax.ShapeDtypeStruct((BATCH, 3), jnp.float32),
    )(e2, W, b.reshape(1, 3))
    return out


# speedup vs baseline: 28.2486x; 1.2090x over previous
import functools

import jax
import jax.numpy as jnp
from jax import lax
from jax.experimental import pallas as pl
from jax.experimental.pallas import tpu as pltpu
from jax.experimental.pallas import tpu_sc as plsc

EMB = 1_000_000
H1 = 16
BATCH = 16384
SEQ = 50
TOKENS = BATCH * SEQ

NC = 2
NS = 16
NW = NC * NS

TOK_PER_W = TOKENS // NW
CHUNK = 128
NCHUNK = TOK_PER_W // CHUNK


K = 10
NSUPER = NCHUNK // K


def _gather_body(idx_hbm, table_hbm, out_hbm, idx_v, rows_v, sem_g, sem_w):
    wid = lax.axis_index("c") * NS + lax.axis_index("s")
    pltpu.sync_copy(idx_hbm.at[pl.ds(wid * NCHUNK, NCHUNK)], idx_v)
    base = wid * TOK_PER_W

    def super_step(s, carry):
        h = lax.rem(s, 2)

        @pl.when(s >= 2)
        def _():
            for b in range(K):
                pltpu.make_async_copy(
                    rows_v.at[h].at[b],
                    out_hbm.at[pl.ds(base, CHUNK)], sem_w).wait()

        handles = [
            pltpu.async_copy(
                table_hbm.at[idx_v.at[s * K + b]], rows_v.at[h].at[b], sem_g)
            for b in range(K)
        ]
        for hd in handles:
            hd.wait()
        for b in range(K):
            tok = base + (s * K + b) * CHUNK
            pltpu.make_async_copy(
                rows_v.at[h].at[b], out_hbm.at[pl.ds(tok, CHUNK)],
                sem_w).start()
        return carry

    lax.fori_loop(0, NSUPER, super_step, 0)
    for h in range(2):
        for b in range(K):
            pltpu.make_async_copy(
                rows_v.at[h].at[b],
                out_hbm.at[pl.ds(base, CHUNK)], sem_w).wait()


_gather = functools.partial(
    pl.kernel,
    out_type=jax.ShapeDtypeStruct((TOKENS, H1), jnp.float32),
    scratch_types=[
        pltpu.VMEM((NCHUNK, CHUNK), jnp.int32),
        pltpu.VMEM((2, K, CHUNK, H1), jnp.float32),
        pltpu.SemaphoreType.DMA,
        pltpu.SemaphoreType.DMA,
    ],
    mesh=plsc.VectorSubcoreMesh(core_axis_name="c", subcore_axis_name="s"),
    compiler_params=pltpu.CompilerParams(use_tc_tiling_on_sc=False),
)(_gather_body)


def _dense_body(e_ref, w_ref, b_ref, o_ref):
    logits = jnp.dot(e_ref[...], w_ref[...],
                     preferred_element_type=jnp.float32) + b_ref[...]
    m = jnp.max(logits, axis=-1, keepdims=True)
    s = jnp.sum(jnp.exp(logits - m), axis=-1, keepdims=True)
    o_ref[...] = logits - m - jnp.log(s)


def kernel(x, table, W, b):
    idx = x.reshape(NW * NCHUNK, CHUNK)
    embeds = _gather(idx, table)
    e2 = embeds.reshape(BATCH, SEQ * H1)

    bm = 2048
    out = pl.pallas_call(
        _dense_body,
        grid=(BATCH // bm,),
        in_specs=[
            pl.BlockSpec((bm, SEQ * H1), lambda i: (i, 0)),
            pl.BlockSpec((SEQ * H1, 3), lambda i: (0, 0)),
            pl.BlockSpec((1, 3), lambda i: (0, 0)),
        ],
        out_specs=pl.BlockSpec((bm, 3), lambda i: (i, 0)),
        out_shape=jax.ShapeDtypeStruct((BATCH, 3), jnp.float32),
    )(e2, W, b.reshape(1, 3))
    return out


# speedup vs baseline: 32.1340x; 1.1375x over previous
import functools

import jax
import jax.numpy as jnp
from jax import lax
from jax.experimental import pallas as pl
from jax.experimental.pallas import tpu as pltpu
from jax.experimental.pallas import tpu_sc as plsc

EMB = 1_000_000
H1 = 16
BATCH = 16384
SEQ = 50

NC = 2
NS = 16
NW = NC * NS

BPW = BATCH // NW
GB = 64
NCH = BPW // GB
TPC = GB * SEQ
NGRP = 8
G = GB // NGRP


def _fused_body(xt_hbm, table_hbm, wr_hbm, prt_hbm,
                wr_v, idx_v, ebuf, obuf, sem_x, sem_g):
    wid = lax.axis_index("c") * NS + lax.axis_index("s")
    b0 = wid * BPW
    pltpu.sync_copy(wr_hbm, wr_v)

    def stage_and_fire(cn):
        hn = lax.rem(cn, 2)
        bc = b0 + cn * GB
        pltpu.async_copy(xt_hbm.at[:, pl.ds(bc, GB)],
                         idx_v.at[hn], sem_x).wait()
        for r in range(SEQ):
            pltpu.make_async_copy(
                table_hbm.at[idx_v.at[hn, r]],
                ebuf.at[hn, pl.ds(r * GB, GB)], sem_g).start()

    def wait_gathers(c):
        h = lax.rem(c, 2)
        for r in range(SEQ):
            pltpu.make_async_copy(
                table_hbm.at[idx_v.at[h, r]],
                ebuf.at[h, pl.ds(r * GB, GB)], sem_g).wait()

    def compute(c):
        h = lax.rem(c, 2)
        bc = b0 + c * GB

        def group(g, carry):
            def lbody(l, accs):
                w0 = wr_v[3 * l]
                w1 = wr_v[3 * l + 1]
                w2 = wr_v[3 * l + 2]
                base = l * GB + g * G
                new = []
                for i in range(G):
                    e = ebuf[h, base + i]
                    new.append(accs[3 * i] + e * w0)
                    new.append(accs[3 * i + 1] + e * w1)
                    new.append(accs[3 * i + 2] + e * w2)
                return tuple(new)

            zero = jnp.zeros((16,), jnp.float32)
            accs = lax.fori_loop(0, SEQ, lbody, (zero,) * (3 * G))
            for i in range(G):
                row = g * G + i
                for c0 in range(3):
                    obuf[row, pl.ds(c0 * 16, 16)] = accs[3 * i + c0]
            return carry

        lax.fori_loop(0, NGRP, group, 0)
        pltpu.sync_copy(obuf, prt_hbm.at[pl.ds(bc, GB)])

    stage_and_fire(0)

    def step(c, carry):
        @pl.when(c < NCH - 1)
        def _():
            stage_and_fire(c + 1)

        wait_gathers(c)
        compute(c)
        return carry

    lax.fori_loop(0, NCH, step, 0)


_fused = functools.partial(
    pl.kernel,
    out_type=jax.ShapeDtypeStruct((BATCH, 48), jnp.float32),
    scratch_types=[
        pltpu.VMEM((152, H1), jnp.float32),
        pltpu.VMEM((2, SEQ, GB), jnp.int32),
        pltpu.VMEM((2, TPC, H1), jnp.float32),
        pltpu.VMEM((GB, 48), jnp.float32),
        pltpu.SemaphoreType.DMA,
        pltpu.SemaphoreType.DMA,
    ],
    mesh=plsc.VectorSubcoreMesh(core_axis_name="c", subcore_axis_name="s"),
    compiler_params=pltpu.CompilerParams(use_tc_tiling_on_sc=False),
)(_fused_body)


def _head_body(p_ref, b_ref, o_ref):
    p = p_ref[...]
    parts = [jnp.sum(p[:, 16 * c0:16 * (c0 + 1)], axis=-1, keepdims=True)
             for c0 in range(3)]
    logits = jnp.concatenate(parts, axis=-1) + b_ref[...]
    m = jnp.max(logits, axis=-1, keepdims=True)
    s = jnp.sum(jnp.exp(logits - m), axis=-1, keepdims=True)
    o_ref[...] = logits - m - jnp.log(s)


def kernel(x, table, W, b):
    xt = x.T
    wr = W.reshape(SEQ, H1, 3).transpose(0, 2, 1).reshape(SEQ * 3, H1)
    wr = jnp.pad(wr, ((0, 2), (0, 0)))
    prt = _fused(xt, table, wr)

    bm = 2048
    out = pl.pallas_call(
        _head_body,
        grid=(BATCH // bm,),
        in_specs=[
            pl.BlockSpec((bm, 48), lambda i: (i, 0)),
            pl.BlockSpec((1, 3), lambda i: (0, 0)),
        ],
        out_specs=pl.BlockSpec((bm, 3), lambda i: (i, 0)),
        out_shape=jax.ShapeDtypeStruct((BATCH, 3), jnp.float32),
    )(prt, b.reshape(1, 3))
    return out


# speedup vs baseline: 32.1895x; 1.0017x over previous
import functools

import jax
import jax.numpy as jnp
from jax import lax
from jax.experimental import pallas as pl
from jax.experimental.pallas import tpu as pltpu
from jax.experimental.pallas import tpu_sc as plsc

EMB = 1_000_000
H1 = 16
BATCH = 16384
SEQ = 50

NC = 2
NS = 16
NW = NC * NS

BPW = BATCH // NW
GB = 64
NCH = BPW // GB
TPC = GB * SEQ
NSTR = TPC // 128
NGRP = 8
G = GB // NGRP


def _fused_body(idx_hbm, table_hbm, wr_hbm, prt_hbm,
                wr_v, idx_v, ebuf, obuf, sem_x, sem_g0, sem_g1):
    wid = lax.axis_index("c") * NS + lax.axis_index("s")
    b0 = wid * BPW
    pltpu.sync_copy(wr_hbm, wr_v)
    gsems = (sem_g0, sem_g1)

    def stage_and_fire(cn):
        h = cn % 2
        pltpu.async_copy(idx_hbm.at[pl.ds((wid * NCH + cn) * NSTR, NSTR)],
                         idx_v.at[h], sem_x).wait()
        for k in range(NSTR):
            pltpu.make_async_copy(
                table_hbm.at[idx_v.at[h, k]],
                ebuf.at[h, pl.ds(k * 128, 128)], gsems[h]).start()

    def wait_gathers(c):
        h = c % 2
        for k in range(NSTR):
            pltpu.make_async_copy(
                table_hbm.at[idx_v.at[h, k]],
                ebuf.at[h, pl.ds(k * 128, 128)], gsems[h]).wait()

    def compute(c):
        h = c % 2
        bc = b0 + c * GB

        def group(g, carry):
            def lbody(l, accs):
                w0 = wr_v[3 * l]
                w1 = wr_v[3 * l + 1]
                w2 = wr_v[3 * l + 2]
                base = lax.rem(l, NSTR) * 128 + (l // NSTR) * GB + g * G
                new = []
                for i in range(G):
                    e = ebuf[h, base + i]
                    new.append(accs[3 * i] + e * w0)
                    new.append(accs[3 * i + 1] + e * w1)
                    new.append(accs[3 * i + 2] + e * w2)
                return tuple(new)

            zero = jnp.zeros((16,), jnp.float32)
            accs = lax.fori_loop(0, SEQ, lbody, (zero,) * (3 * G))
            for i in range(G):
                row = g * G + i
                for c0 in range(3):
                    obuf[row, pl.ds(c0 * 16, 16)] = accs[3 * i + c0]
            return carry

        lax.fori_loop(0, NGRP, group, 0)
        pltpu.sync_copy(obuf, prt_hbm.at[pl.ds(bc, GB)])

    stage_and_fire(0)
    for c in range(NCH):
        if c < NCH - 1:
            stage_and_fire(c + 1)
        wait_gathers(c)
        compute(c)


_fused = functools.partial(
    pl.kernel,
    out_type=jax.ShapeDtypeStruct((BATCH, 48), jnp.float32),
    scratch_types=[
        pltpu.VMEM((152, H1), jnp.float32),
        pltpu.VMEM((2, NSTR, 128), jnp.int32),
        pltpu.VMEM((2, TPC, H1), jnp.float32),
        pltpu.VMEM((GB, 48), jnp.float32),
        pltpu.SemaphoreType.DMA,
        pltpu.SemaphoreType.DMA,
        pltpu.SemaphoreType.DMA,
    ],
    mesh=plsc.VectorSubcoreMesh(core_axis_name="c", subcore_axis_name="s"),
    compiler_params=pltpu.CompilerParams(use_tc_tiling_on_sc=False),
)(_fused_body)


def _prep_body(x_ref, o_ref):
    x8 = x_ref[...].reshape(NCH, GB, SEQ)
    t = x8.transpose(0, 2, 1)
    o_ref[...] = jnp.concatenate(
        [t[:, :NSTR, :], t[:, NSTR:, :]], axis=-1).reshape(NCH * NSTR, 128)


def _head_body(p_ref, b_ref, o_ref):
    p = p_ref[...]
    parts = [jnp.sum(p[:, 16 * c0:16 * (c0 + 1)], axis=-1, keepdims=True)
             for c0 in range(3)]
    logits = jnp.concatenate(parts, axis=-1) + b_ref[...]
    m = jnp.max(logits, axis=-1, keepdims=True)
    s = jnp.sum(jnp.exp(logits - m), axis=-1, keepdims=True)
    o_ref[...] = logits - m - jnp.log(s)


def kernel(x, table, W, b):
    idx = pl.pallas_call(
        _prep_body,
        grid=(NW,),
        in_specs=[pl.BlockSpec((BPW, SEQ), lambda i: (i, 0))],
        out_specs=pl.BlockSpec((NCH * NSTR, 128), lambda i: (i, 0)),
        out_shape=jax.ShapeDtypeStruct((BATCH // GB * NSTR, 128), jnp.int32),
    )(x)
    wr = W.reshape(SEQ, H1, 3).transpose(0, 2, 1).reshape(SEQ * 3, H1)
    wr = jnp.pad(wr, ((0, 2), (0, 0)))
    prt = _fused(idx, table, wr)

    bm = 2048
    out = pl.pallas_call(
        _head_body,
        grid=(BATCH // bm,),
        in_specs=[
            pl.BlockSpec((bm, 48), lambda i: (i, 0)),
            pl.BlockSpec((1, 3), lambda i: (0, 0)),
        ],
        out_specs=pl.BlockSpec((bm, 3), lambda i: (i, 0)),
        out_shape=jax.ShapeDtypeStruct((BATCH, 3), jnp.float32),
    )(prt, b.reshape(1, 3))
    return out
